# baseline (speedup 1.0000x reference)
import jax
import jax.numpy as jnp
from jax.experimental import pallas as pl
from jax.experimental.pallas import tpu as pltpu


def _cd_kernel(src_ref, tgt_ref, out_ref, sa_ref, ta_ref):
    i = pl.program_id(0)
    jb = pl.program_id(1)

    @pl.when(jnp.logical_and(i == 0, jb == 0))
    def _augment():
        for c in range(16):
            s = src_ref[c]
            t = tgt_ref[c]
            ns = jnp.sum(s * s, axis=1, keepdims=True)
            nt = jnp.sum(t * t, axis=1, keepdims=True)
            ns_hi = ns.astype(jnp.bfloat16).astype(jnp.float32)
            ns_lo = ns - ns_hi
            nt_hi = nt.astype(jnp.bfloat16).astype(jnp.float32)
            nt_lo = nt - nt_hi
            ones = jnp.ones_like(ns)
            sa = jnp.concatenate(
                [-2.0 * s, ns_hi, ns_lo, ones, ones, ns * 0.0], axis=1)
            ta = jnp.concatenate(
                [t, ones, ones, nt_hi, nt_lo, nt * 0.0], axis=1)
            sa_ref[c] = sa.astype(jnp.bfloat16)
            ta_ref[c] = ta.astype(jnp.bfloat16)

    for si in range(2):
        s = sa_ref[i * 2 + si]
        for j in range(8):
            t = ta_ref[jb * 8 + j]
            fwd_sum = 0.0
            colmin = None
            for r in range(4):
                d2 = jax.lax.dot_general(
                    s[r * 512:(r + 1) * 512], t, (((1,), (1,)), ((), ())),
                    preferred_element_type=jnp.float32,
                )
                rowmin = jnp.min(d2, axis=1)
                cmin = jnp.min(d2, axis=0)
                colmin = cmin if colmin is None else jnp.minimum(colmin, cmin)
                fwd_sum += jnp.sum(jnp.sqrt(jnp.maximum(rowmin, 1e-12)))
            fwd = fwd_sum / 2048.0
            bwd = jnp.mean(jnp.sqrt(jnp.maximum(colmin, 1e-12)))
            out_ref[si, j] = jnp.full((1, 128), fwd + bwd, dtype=jnp.float32)


@jax.jit
def kernel(source, target):
    b, n, d = source.shape
    cd = pl.pallas_call(
        _cd_kernel,
        grid=(b // 2, b // 8),
        in_specs=[
            pl.BlockSpec((b, n, d), lambda i, jb: (0, 0, 0)),
            pl.BlockSpec((b, n, d), lambda i, jb: (0, 0, 0)),
        ],
        out_specs=pl.BlockSpec((2, 8, 1, 128), lambda i, jb: (i, jb, 0, 0)),
        out_shape=jax.ShapeDtypeStruct((b, b, 1, 128), jnp.float32),
        scratch_shapes=[
            pltpu.VMEM((b, n, 8), jnp.bfloat16),
            pltpu.VMEM((b, n, 8), jnp.bfloat16),
        ],
    )(source, target)[:, :, 0, 0]

    return jnp.mean(jnp.min(cd, axis=1))


# speedup vs baseline: 1.5245x; 1.5245x over previous
import jax
import jax.numpy as jnp
from jax.experimental import pallas as pl
from jax.experimental.pallas import tpu as pltpu


def _cd_kernel(src_ref, tgt_ref, out_ref, sa_ref, ta_ref, minacc_ref):
    i = pl.program_id(0)
    jb = pl.program_id(1)

    @pl.when(jnp.logical_and(i == 0, jb == 0))
    def _augment():
        for c in range(16):
            s = src_ref[c]
            t = tgt_ref[c]
            ns = jnp.sum(s * s, axis=1, keepdims=True)
            nt = jnp.sum(t * t, axis=1, keepdims=True)
            ns_hi = ns.astype(jnp.bfloat16).astype(jnp.float32)
            ns_lo = ns - ns_hi
            nt_hi = nt.astype(jnp.bfloat16).astype(jnp.float32)
            nt_lo = nt - nt_hi
            ones = jnp.ones_like(ns)
            sa = jnp.concatenate(
                [-2.0 * s, ns_hi, ns_lo, ones, ones, ns * 0.0], axis=1)
            ta = jnp.concatenate(
                [t, ones, ones, nt_hi, nt_lo, nt * 0.0], axis=1)
            sa_ref[c] = sa.astype(jnp.bfloat16)
            ta_ref[c] = ta.astype(jnp.bfloat16)

    @pl.when(jnp.logical_and(i == 0, jb == 0))
    def _init_minacc():
        minacc_ref[...] = jnp.full((16, 128), jnp.inf, dtype=jnp.float32)

    s = sa_ref[i]
    cds = []
    for j in range(8):
        t = ta_ref[jb * 8 + j]
        fwd_sum = 0.0
        colmin = None
        for r in range(4):
            d2 = jax.lax.dot_general(
                s[r * 512:(r + 1) * 512], t, (((1,), (1,)), ((), ())),
                preferred_element_type=jnp.float32,
            )
            rowmin = jnp.min(d2, axis=1)
            cmin = jnp.min(d2, axis=0)
            colmin = cmin if colmin is None else jnp.minimum(colmin, cmin)
            fwd_sum += jnp.sum(jnp.sqrt(jnp.maximum(rowmin, 1e-12)))
        fwd = fwd_sum / 2048.0
        bwd = jnp.mean(jnp.sqrt(jnp.maximum(colmin, 1e-12)))
        cds.append(fwd + bwd)

    blockmin = cds[0]
    for c in cds[1:]:
        blockmin = jnp.minimum(blockmin, c)
    row = minacc_ref[pl.ds(i, 1), :]
    minacc_ref[pl.ds(i, 1), :] = jnp.minimum(
        row, jnp.full((1, 128), blockmin, dtype=jnp.float32))

    @pl.when(jnp.logical_and(i == 15, jb == 1))
    def _finalize():
        out_ref[...] = jnp.broadcast_to(
            jnp.mean(minacc_ref[:, 0:1]), (8, 128)).astype(jnp.float32)


@jax.jit
def kernel(source, target):
    b, n, d = source.shape
    cd = pl.pallas_call(
        _cd_kernel,
        grid=(b, b // 8),
        in_specs=[
            pl.BlockSpec((b, n, d), lambda i, jb: (0, 0, 0)),
            pl.BlockSpec((b, n, d), lambda i, jb: (0, 0, 0)),
        ],
        out_specs=pl.BlockSpec((8, 128), lambda i, jb: (0, 0)),
        out_shape=jax.ShapeDtypeStruct((8, 128), jnp.float32),
        scratch_shapes=[
            pltpu.VMEM((b, n, 8), jnp.bfloat16),
            pltpu.VMEM((b, n, 8), jnp.bfloat16),
            pltpu.VMEM((b, 128), jnp.float32),
        ],
    )(source, target)

    return cd[0, 0]
